# baseline (speedup 1.0000x reference)
import functools
import jax
import jax.numpy as jnp
from jax import lax
from jax.experimental import pallas as pl
from jax.experimental.pallas import tpu as pltpu
from jax.experimental.pallas import tpu_sc as plsc

H = 512
W = 512
C = 17
P = H * W
NW = 32
PPW = P // NW
CH = 2048
ROUNDS = PPW // CH
L = 16


def _sc_body(hm_hbm, off_hbm, out_hbm, hm_v, off_v, out_v):
    wid = lax.axis_index("s") * 2 + lax.axis_index("c")
    iota = lax.iota(jnp.int32, L)
    iota17 = iota * C
    iota34 = iota * (2 * C)
    iota4 = iota * 4

    for r in range(ROUNDS):
        base_p = wid * PPW + r * CH
        pltpu.sync_copy(hm_hbm.at[pl.ds(base_p * C, CH * C)], hm_v)
        pltpu.sync_copy(off_hbm.at[pl.ds(base_p * 2 * C, CH * 2 * C)], off_v)

        def body(j, _):
            hbase = j * (L * C) + iota17
            best = plsc.load_gather(hm_v, [hbase])
            bestc = jnp.zeros((L,), jnp.int32)
            for c in range(1, C):
                v = plsc.load_gather(hm_v, [hbase + c])
                gt = v > best
                best = jnp.where(gt, v, best)
                bestc = jnp.where(gt, jnp.full((L,), c, jnp.int32), bestc)
            score = 1.0 / (1.0 + jnp.exp(-best))

            obase = j * (L * 2 * C) + iota34 + bestc
            y_off = plsc.load_gather(off_v, [obase])
            x_off = plsc.load_gather(off_v, [obase + C])

            p_glob = base_p + j * L + iota
            py = (p_glob >> 9).astype(jnp.float32)
            px = (p_glob & 511).astype(jnp.float32)
            xv = (px * 4.0 + x_off).astype(jnp.int32).astype(jnp.float32)
            yv = (py * 4.0 + y_off).astype(jnp.int32).astype(jnp.float32)

            sbase = j * (L * 4) + iota4
            plsc.store_scatter(out_v, [sbase], bestc.astype(jnp.float32))
            plsc.store_scatter(out_v, [sbase + 1], score)
            plsc.store_scatter(out_v, [sbase + 2], xv)
            plsc.store_scatter(out_v, [sbase + 3], yv)
            return 0

        lax.fori_loop(0, CH // L, body, 0)
        pltpu.sync_copy(out_v, out_hbm.at[pl.ds(base_p * 4, CH * 4)])


def kernel(heatmaps_input, offsets_input):
    hm = heatmaps_input.reshape(H * W * C)
    off = offsets_input.reshape(H * W * 2 * C)
    k = functools.partial(
        pl.kernel,
        out_type=jax.ShapeDtypeStruct((P * 4,), jnp.float32),
        scratch_types=[
            pltpu.VMEM((CH * C,), jnp.float32),
            pltpu.VMEM((CH * 2 * C,), jnp.float32),
            pltpu.VMEM((CH * 4,), jnp.float32),
        ],
        mesh=plsc.VectorSubcoreMesh(core_axis_name="c", subcore_axis_name="s"),
        compiler_params=pltpu.CompilerParams(
            use_tc_tiling_on_sc=False, needs_layout_passes=False
        ),
    )(_sc_body)
    out = k(hm, off)
    return out.reshape(1, P, 4)
